# baseline (speedup 1.0000x reference)
import jax
import jax.numpy as jnp
from jax import lax
from jax.experimental import pallas as pl
from jax.experimental.pallas import tpu as pltpu
from jax.experimental.pallas import tpu_sc as plsc

VOCAB = 1_000_000
L = 200
D = 32
B = 4096

NC, NS = 2, 16
NW = NC * NS
SEQ_W = B // NW

NSEQ = 8
SPLITS = ((0, 104), (104, 96))
NCHUNK = SEQ_W // NSEQ
NBUF = 2


def _emb_body(x_hbm, tok_hbm, pos_hbm, out_hbm, idx_v, rows_v, pos_v, gsem, wsem):
    cid = lax.axis_index("c")
    sid = lax.axis_index("s")
    wid = sid * NC + cid
    base = wid * SEQ_W

    pltpu.sync_copy(pos_hbm, pos_v)

    def fire(g):
        p = lax.rem(g, NBUF)
        pltpu.sync_copy(x_hbm.at[pl.ds(base + g * NSEQ, NSEQ)], idx_v.at[p])
        for s in range(NSEQ):
            for lo, sz in SPLITS:
                pltpu.async_copy(
                    tok_hbm.at[idx_v.at[p, s, pl.ds(lo, sz)]],
                    rows_v.at[p, s, pl.ds(lo, sz)],
                    gsem.at[p],
                )

    fire(0)

    def chunk(g, _):
        p = lax.rem(g, NBUF)

        @pl.when(g + 1 < NCHUNK)
        def _fire_next():
            p1 = lax.rem(g + 1, NBUF)

            @pl.when(g + 1 >= NBUF)
            def _wait_writeback():
                pltpu.make_async_copy(
                    rows_v.at[p1], out_hbm.at[pl.ds(0, NSEQ)], wsem.at[p1]
                ).wait()

            fire(g + 1)

        pltpu.make_async_copy(
            out_hbm.at[pl.ds(0, NSEQ)], rows_v.at[p], gsem.at[p]
        ).wait()

        for s in range(NSEQ):

            @plsc.parallel_loop(0, L, unroll=8)
            def _add_pos(r):
                plsc.addupdate(
                    rows_v.at[p, s, r, pl.ds(0, 16)], pos_v[r, pl.ds(0, 16)]
                )
                plsc.addupdate(
                    rows_v.at[p, s, r, pl.ds(16, 16)], pos_v[r, pl.ds(16, 16)]
                )

        pltpu.async_copy(
            rows_v.at[p], out_hbm.at[pl.ds(base + g * NSEQ, NSEQ)], wsem.at[p]
        )
        return 0

    lax.fori_loop(0, NCHUNK, chunk, 0)

    for b in range(NBUF):
        pltpu.make_async_copy(
            rows_v.at[b], out_hbm.at[pl.ds(0, NSEQ)], wsem.at[b]
        ).wait()


@jax.jit
def _emb(x, token_table, pos_table):
    mesh = plsc.VectorSubcoreMesh(core_axis_name="c", subcore_axis_name="s")
    return pl.kernel(
        _emb_body,
        out_type=jax.ShapeDtypeStruct((B, L, D), jnp.float32),
        mesh=mesh,
        compiler_params=pltpu.CompilerParams(use_tc_tiling_on_sc=False),
        scratch_types=[
            pltpu.VMEM((NBUF, NSEQ, L), jnp.int32),
            pltpu.VMEM((NBUF, NSEQ, L, D), jnp.float32),
            pltpu.VMEM((L, D), jnp.float32),
            pltpu.SemaphoreType.DMA((NBUF,)),
            pltpu.SemaphoreType.DMA((NBUF,)),
        ],
    )(x, token_table, pos_table)


def kernel(x, token_table, pos_table):
    return _emb(x.astype(jnp.int32), token_table, pos_table)


# speedup vs baseline: 1.1575x; 1.1575x over previous
import jax
import jax.numpy as jnp
from jax import lax
from jax.experimental import pallas as pl
from jax.experimental.pallas import tpu as pltpu
from jax.experimental.pallas import tpu_sc as plsc

VOCAB = 1_000_000
L = 200
D = 32
B = 4096

NC, NS = 2, 16
NW = NC * NS
BW = B // NW

PL = 4
G = L // PL
NBUF = 2
DH, DL = D // 8, 8


def _emb_body(xt_hbm, tok_hbm, pos_hbm, out_hbm, idx_v, rows_v, tblk_v, pos_v, gsem, wsem):
    cid = lax.axis_index("c")
    sid = lax.axis_index("s")
    wid = sid * NC + cid
    b0 = wid * BW

    pltpu.sync_copy(pos_hbm, pos_v)

    iota = lax.iota(jnp.int32, 16)
    row_ids = [iota + (blk * 16) for blk in range(BW // 16)]

    def fire(g):
        p = lax.rem(g, NBUF)
        pltpu.sync_copy(
            xt_hbm.at[pl.ds(g * PL, PL), pl.ds(b0, BW)], idx_v.at[p]
        )
        for j in range(PL):
            pltpu.async_copy(
                tok_hbm.at[idx_v.at[p, j]], rows_v.at[p, j], gsem.at[p]
            )

    fire(0)

    def chunk(g, _):
        p = lax.rem(g, NBUF)

        @pl.when(g + 1 < G)
        def _fire_next():
            fire(g + 1)

        for j in range(PL):
            pltpu.make_async_copy(
                tok_hbm.at[pl.ds(0, BW)], rows_v.at[p, j], gsem.at[p]
            ).wait()

        @pl.when(g >= NBUF)
        def _wait_writeback():
            for j in range(PL):
                for dh in range(DH):
                    pltpu.make_async_copy(
                        tblk_v.at[p, j, dh], out_hbm.at[0, dh, 0], wsem.at[p]
                    ).wait()

        for j in range(PL):
            lpos = g * PL + j
            rows = rows_v.at[p, j]
            prow = pos_v.at[lpos]

            @plsc.parallel_loop(0, D, unroll=4)
            def _transpose(d):
                dh = lax.shift_right_logical(d, 3)
                dl = lax.bitwise_and(d, 7)
                dsplat = jnp.broadcast_to(d, (16,))
                psplat = plsc.load_gather(prow, [dsplat])
                for blk in range(BW // 16):
                    val = plsc.load_gather(rows, [row_ids[blk], dsplat])
                    tblk_v[p, j, dh, dl, pl.ds(blk * 16, 16)] = val + psplat

        for j in range(PL):
            for dh in range(DH):
                pltpu.async_copy(
                    tblk_v.at[p, j, dh],
                    out_hbm.at[g * PL + j, dh, wid],
                    wsem.at[p],
                )
        return 0

    lax.fori_loop(0, G, chunk, 0)

    for b in range(NBUF):
        for j in range(PL):
            for dh in range(DH):
                pltpu.make_async_copy(
                    tblk_v.at[b, j, dh], out_hbm.at[0, dh, 0], wsem.at[b]
                ).wait()


@jax.jit
def _emb(xt, token_table, pos_table):
    mesh = plsc.VectorSubcoreMesh(core_axis_name="c", subcore_axis_name="s")
    return pl.kernel(
        _emb_body,
        out_type=jax.ShapeDtypeStruct((L, DH, NW, DL, BW), jnp.float32),
        mesh=mesh,
        compiler_params=pltpu.CompilerParams(
            use_tc_tiling_on_sc=False, needs_layout_passes=False
        ),
        scratch_types=[
            pltpu.VMEM((NBUF, PL, BW), jnp.int32),
            pltpu.VMEM((NBUF, PL, BW, D), jnp.float32),
            pltpu.VMEM((NBUF, PL, DH, DL, BW), jnp.float32),
            pltpu.VMEM((L, D), jnp.float32),
            pltpu.SemaphoreType.DMA((NBUF,)),
            pltpu.SemaphoreType.DMA((NBUF,)),
        ],
    )(xt, token_table, pos_table)


def kernel(x, token_table, pos_table):
    xt = jnp.transpose(x.astype(jnp.int32))
    out5 = _emb(xt, token_table, pos_table)
    return jnp.transpose(out5, (2, 4, 0, 1, 3)).reshape(B, L, D)


# speedup vs baseline: 1.6525x; 1.4277x over previous
import jax
import jax.numpy as jnp
from jax import lax
from jax.experimental import pallas as pl
from jax.experimental.pallas import tpu as pltpu
from jax.experimental.pallas import tpu_sc as plsc

VOCAB = 1_000_000
L = 200
D = 32
B = 4096

NC, NS = 2, 16
NW = NC * NS
BW = B // NW

LH, LL = L // 8, 8
DH, DL = D // 8, 8
PL = 4
G = L // PL
NBUF = 2
BWP = BW + 1


def _emb_body(x4_hbm, tok_hbm, pos_hbm, out_hbm, idx_v, rows_v, tblk_v, pos_v, gsem, wsem):
    cid = lax.axis_index("c")
    sid = lax.axis_index("s")
    wid = sid * NC + cid

    pltpu.sync_copy(pos_hbm, pos_v)

    iota = lax.iota(jnp.int32, 16)

    def fire(g):
        p = lax.rem(g, NBUF)
        lh = lax.div(g, LL // PL)
        ll0 = lax.rem(g, LL // PL) * PL
        pltpu.sync_copy(x4_hbm.at[lh, wid, pl.ds(ll0, PL)], idx_v.at[p])
        for j in range(PL):
            pltpu.async_copy(
                tok_hbm.at[idx_v.at[p, j]], rows_v.at[p, j], gsem.at[p]
            )

    fire(0)

    def chunk(g, _):
        p = lax.rem(g, NBUF)

        @pl.when(g + 1 < G)
        def _fire_next():
            fire(g + 1)

        for j in range(PL):
            pltpu.make_async_copy(
                tok_hbm.at[pl.ds(0, BW)], rows_v.at[p, j], gsem.at[p]
            ).wait()

        @pl.when(g >= NBUF)
        def _wait_writeback():
            for j in range(PL):
                for dh in range(DH):
                    pltpu.make_async_copy(
                        tblk_v.at[p, j, pl.ds(dh * DL, DL), pl.ds(0, BW)],
                        out_hbm.at[0, dh, 0],
                        wsem.at[p],
                    ).wait()

        for j in range(PL):
            lpos = g * PL + j
            prow0 = pos_v[lpos, pl.ds(0, 16)]
            prow1 = pos_v[lpos, pl.ds(16, 16)]
            rows = rows_v.at[p, j]
            tblk = tblk_v.at[p, j]

            @plsc.parallel_loop(0, BW, unroll=8)
            def _transpose(b):
                bsplat = jnp.broadcast_to(b, (16,))
                v0 = rows[b, pl.ds(0, 16)] + prow0
                v1 = rows[b, pl.ds(16, 16)] + prow1
                plsc.store_scatter(tblk, [iota, bsplat], v0)
                plsc.store_scatter(tblk, [iota + 16, bsplat], v1)

        for j in range(PL):
            for dh in range(DH):
                pltpu.async_copy(
                    tblk_v.at[p, j, pl.ds(dh * DL, DL), pl.ds(0, BW)],
                    out_hbm.at[g * PL + j, dh, wid],
                    wsem.at[p],
                )
        return 0

    lax.fori_loop(0, G, chunk, 0)

    for b in range(NBUF):
        for j in range(PL):
            for dh in range(DH):
                pltpu.make_async_copy(
                    tblk_v.at[b, j, pl.ds(dh * DL, DL), pl.ds(0, BW)],
                    out_hbm.at[0, dh, 0],
                    wsem.at[b],
                ).wait()


@jax.jit
def _emb(x4, token_table, pos_table):
    mesh = plsc.VectorSubcoreMesh(core_axis_name="c", subcore_axis_name="s")
    return pl.kernel(
        _emb_body,
        out_type=jax.ShapeDtypeStruct((L, DH, NW, DL, BW), jnp.float32),
        mesh=mesh,
        compiler_params=pltpu.CompilerParams(
            use_tc_tiling_on_sc=False, needs_layout_passes=False
        ),
        scratch_types=[
            pltpu.VMEM((NBUF, PL, BW), jnp.int32),
            pltpu.VMEM((NBUF, PL, BW, D), jnp.float32),
            pltpu.VMEM((NBUF, PL, D, BWP), jnp.float32),
            pltpu.VMEM((L, D), jnp.float32),
            pltpu.SemaphoreType.DMA((NBUF,)),
            pltpu.SemaphoreType.DMA((NBUF,)),
        ],
    )(x4, token_table, pos_table)


def kernel(x, token_table, pos_table):
    x4 = (
        x.astype(jnp.int32)
        .reshape(NW, BW, LH, LL)
        .transpose(2, 0, 3, 1)
    )
    out5 = _emb(x4, token_table, pos_table)
    return jnp.transpose(out5, (2, 4, 0, 1, 3)).reshape(B, L, D)
